# baseline (speedup 1.0000x reference)
import functools

import jax
import jax.numpy as jnp
from jax import lax
from jax.experimental import pallas as pl
from jax.experimental.pallas import tpu as pltpu
from jax.experimental.pallas import tpu_sc as plsc


def _sc_gather(table, idx):
    V, D = table.shape
    (B,) = idx.shape
    info = plsc.get_sparse_core_info()
    NW = info.num_cores * info.num_subcores
    b_per_w = B // NW
    C = 40
    sizes = [C] * (b_per_w // C)
    if b_per_w % C:
        sizes.append(b_per_w % C)
    offs = [sum(sizes[:i]) for i in range(len(sizes))]
    n_chunks = len(sizes)
    mesh = plsc.VectorSubcoreMesh(core_axis_name="c", subcore_axis_name="s")

    NB = 3

    @functools.partial(
        pl.kernel,
        mesh=mesh,
        out_type=jax.ShapeDtypeStruct((B, D), jnp.float32),
        scratch_types=[
            pltpu.VMEM((b_per_w,), jnp.int32),
        ]
        + [pltpu.VMEM((C, D), jnp.float32) for _ in range(NB)]
        + [pltpu.SemaphoreType.DMA for _ in range(2 * NB)],
    )
    def k(table_hbm, idx_hbm, out_hbm, idx_v, *bufs_and_sems):
        bufs = bufs_and_sems[:NB]
        gsems = bufs_and_sems[NB : 2 * NB]
        wsems = bufs_and_sems[2 * NB : 3 * NB]
        wid = lax.axis_index("s") * info.num_cores + lax.axis_index("c")
        base = wid * b_per_w
        pltpu.sync_copy(idx_hbm.at[pl.ds(base, b_per_w)], idx_v)
        pend_g, pend_w = {}, {}
        for c in range(n_chunks + 1):
            if c < n_chunks:
                b = c % NB
                if c >= NB:
                    pend_w.pop(c - NB).wait()
                pend_g[c] = pltpu.async_copy(
                    table_hbm.at[idx_v.at[pl.ds(offs[c], sizes[c])]],
                    bufs[b].at[pl.ds(0, sizes[c])],
                    gsems[b],
                )
            if c >= 1:
                pend_g.pop(c - 1).wait()
                pend_w[c - 1] = pltpu.async_copy(
                    bufs[(c - 1) % NB].at[pl.ds(0, sizes[c - 1])],
                    out_hbm.at[pl.ds(base + offs[c - 1], sizes[c - 1])],
                    wsems[(c - 1) % NB],
                )
        for c in sorted(pend_w):
            pend_w[c].wait()

    return k(table, idx)


def _tc_compute(g, u, interpret=False):
    B, A = g.shape
    R = 2048
    grid = B // R

    def body(g_ref, u_ref, act_ref, sp_ref, lp_ref):
        p = g_ref[...]
        e = -jnp.log(u_ref[...])
        r = p / e
        a = jnp.argmax(r, axis=-1)
        s = jnp.sum(p, axis=-1)
        cols = lax.broadcasted_iota(jnp.int32, p.shape, 1)
        v = jnp.sum(jnp.where(cols == a[:, None], p, 0.0), axis=-1)
        ratio = v / s
        act_ref[...] = a[:, None]
        sp_ref[...] = ratio[:, None]
        lp_ref[...] = jnp.log(ratio)[:, None]

    acts, sps, lps = pl.pallas_call(
        body,
        grid=(grid,),
        in_specs=[
            pl.BlockSpec((R, A), lambda i: (i, 0)),
            pl.BlockSpec((R, A), lambda i: (i, 0)),
        ],
        out_specs=[
            pl.BlockSpec((R, 1), lambda i: (i, 0)),
            pl.BlockSpec((R, 1), lambda i: (i, 0)),
            pl.BlockSpec((R, 1), lambda i: (i, 0)),
        ],
        out_shape=[
            jax.ShapeDtypeStruct((B, 1), jnp.int32),
            jax.ShapeDtypeStruct((B, 1), jnp.float32),
            jax.ShapeDtypeStruct((B, 1), jnp.float32),
        ],
        interpret=interpret,
    )(g, u)
    return acts[:, 0], sps[:, 0], lps[:, 0]


def _unused_tc_gather_compute(table, idx, u, interpret=False):
    V, A = table.shape
    (B,) = idx.shape
    R = 512
    grid = B // R

    def body(idx_ref, table_ref, u_ref, act_ref, sp_ref, lp_ref, rows, sem0, sem1):
        i = pl.program_id(0)
        sems = (sem0, sem1)

        def issue(step, slot):
            def one(r, _):
                iv = idx_ref[step * R + r]
                pltpu.make_async_copy(
                    table_ref.at[pl.ds(iv, 1)],
                    rows.at[slot, pl.ds(r, 1)],
                    sems[slot],
                ).start()
                return 0

            lax.fori_loop(0, R, one, 0)

        @pl.when(i == 0)
        def _():
            issue(i, 0)

        slot = lax.rem(i, 2)
        nslot = lax.rem(i + 1, 2)

        @pl.when(i + 1 < grid)
        def _():
            issue(i + 1, nslot)

        pltpu.make_async_copy(
            table_ref.at[pl.ds(0, R)], rows.at[slot], sems[slot]
        ).wait()

        p = rows[slot]
        e = -jnp.log(u_ref[...])
        r_ = p / e
        a = jnp.argmax(r_, axis=-1)
        s = jnp.sum(p, axis=-1)
        cols = lax.broadcasted_iota(jnp.int32, p.shape, 1)
        v = jnp.sum(jnp.where(cols == a[:, None], p, 0.0), axis=-1)
        ratio = v / s
        act_ref[...] = a[:, None]
        sp_ref[...] = ratio[:, None]
        lp_ref[...] = jnp.log(ratio)[:, None]

    acts, sps, lps = pl.pallas_call(
        body,
        grid_spec=pltpu.PrefetchScalarGridSpec(
            num_scalar_prefetch=1,
            grid=(grid,),
            in_specs=[
                pl.BlockSpec(memory_space=pl.ANY),
                pl.BlockSpec((R, A), lambda i, idx: (i, 0)),
            ],
            out_specs=[
                pl.BlockSpec((R, 1), lambda i, idx: (i, 0)),
                pl.BlockSpec((R, 1), lambda i, idx: (i, 0)),
                pl.BlockSpec((R, 1), lambda i, idx: (i, 0)),
            ],
            scratch_shapes=[
                pltpu.VMEM((2, R, A), jnp.float32),
                pltpu.SemaphoreType.DMA,
                pltpu.SemaphoreType.DMA,
            ],
        ),
        out_shape=[
            jax.ShapeDtypeStruct((B, 1), jnp.int32),
            jax.ShapeDtypeStruct((B, 1), jnp.float32),
            jax.ShapeDtypeStruct((B, 1), jnp.float32),
        ],
        interpret=interpret,
    )(idx, table, u)
    return acts[:, 0], sps[:, 0], lps[:, 0]


def kernel(probs_table, state_idx, u):
    g = _sc_gather(probs_table, state_idx)
    return _tc_compute(g, u)
